# baseline (speedup 1.0000x reference)
import functools

import jax
import jax.numpy as jnp
from jax import lax
from jax.experimental import pallas as pl
from jax.experimental.pallas import tpu as pltpu
from jax.experimental.pallas import tpu_sc as plsc

N = 10000
E = 320000
DI = 128
DH = 256
CH = 128
NI = 40
EPAD = 327680
NACC = 10240
NSUB = 16
NCORE = 2
R = 1000

_MESH = plsc.VectorSubcoreMesh(core_axis_name="c", subcore_axis_name="s")
_ZV = functools.partial(jnp.zeros, (16,), jnp.float32)


def _deg_body(dst_hbm, out_hbm, acc, di, ones, sz):
    cid = lax.axis_index("c")
    sid = lax.axis_index("s")
    nck = EPAD // (NCORE * NSUB) // CH
    zrows = NACC // NSUB

    def fill(i, _):
        for c8 in range(DI // 16):
            ones[i, pl.ds(c8 * 16, 16)] = _ZV()
        return 0
    lax.fori_loop(0, CH, fill, 0)

    zd = [pltpu.async_copy(ones, acc.at[pl.ds(sid * zrows + i * CH, CH)], sz)
          for i in range(zrows // CH)]
    row0 = (cid * NSUB + sid) * nck
    pltpu.sync_copy(dst_hbm.at[pl.ds(row0, nck)], di)
    for d in zd:
        d.wait()

    one = jnp.ones((16,), jnp.float32)

    def refill(i, _):
        for c8 in range(DI // 16):
            ones[i, pl.ds(c8 * 16, 16)] = one
        return 0
    lax.fori_loop(0, CH, refill, 0)
    plsc.subcore_barrier()

    FL = 8
    def step(c, _):
        ws = [pltpu.async_copy(ones, acc.at[di.at[c * FL + k]], sz, add=True)
              for k in range(FL)]
        for w in ws:
            w.wait()
        return 0
    lax.fori_loop(0, nck // FL, step, 0)

    plsc.subcore_barrier()
    pltpu.sync_copy(acc.at[pl.ds(sid * zrows, zrows)],
                    out_hbm.at[cid, pl.ds(sid * zrows, zrows)])


_deg_call = pl.kernel(
    _deg_body,
    out_type=jax.ShapeDtypeStruct((NCORE, NACC, DI), jnp.float32),
    mesh=_MESH,
    scratch_types=[
        pltpu.VMEM_SHARED((NACC, DI), jnp.float32),
        pltpu.VMEM((EPAD // (NCORE * NSUB) // CH, CH), jnp.int32),
        pltpu.VMEM((CH, DI), jnp.float32),
        pltpu.SemaphoreType.DMA,
    ],
)


def _agg_body(nck, b0r, b1r, mult, tbl, src_hbm, dst_hbm, out_hbm,
              acc, si, di, ra, rb, sz, sga, sgb, ssa, ssb):
    cid = lax.axis_index("c")
    sid = lax.axis_index("s")
    zrows = NACC // NSUB

    def zrow(i, _):
        for c8 in range(DI // 16):
            ra[i, pl.ds(c8 * 16, 16)] = _ZV()
        return 0
    lax.fori_loop(0, CH, zrow, 0)

    zd = [pltpu.async_copy(ra, acc.at[pl.ds(sid * zrows + i * CH, CH)], sz)
          for i in range(zrows // CH)]
    row0 = jnp.where(cid == 0, b0r, b1r) + sid * nck
    off = cid * (mult * N)
    for d in zd:
        d.wait()
    plsc.subcore_barrier()

    def stage(s, _):
        pltpu.sync_copy(src_hbm.at[pl.ds(row0 + s * NI, NI)], si)
        pltpu.sync_copy(dst_hbm.at[pl.ds(row0 + s * NI, NI)], di)
        if mult:
            def shift(r, _):
                for c8 in range(CH // 16):
                    si[r, pl.ds(c8 * 16, 16)] = (
                        si[r, pl.ds(c8 * 16, 16)] + off)
                return 0
            lax.fori_loop(0, NI, shift, 0)
        da = pltpu.async_copy(tbl.at[si.at[0]], ra, sga)
        wb = None
        for p in range(NI // 2):
            if wb is not None:
                wb.wait()
            db = pltpu.async_copy(tbl.at[si.at[2 * p + 1]], rb, sgb)
            da.wait()
            wa = pltpu.async_copy(ra, acc.at[di.at[2 * p]], ssa, add=True)
            db.wait()
            wb = pltpu.async_copy(rb, acc.at[di.at[2 * p + 1]], ssb, add=True)
            wa.wait()
            if p + 1 < NI // 2:
                da = pltpu.async_copy(tbl.at[si.at[2 * p + 2]], ra, sga)
        wb.wait()
        return 0
    lax.fori_loop(0, nck // NI, stage, 0)

    plsc.subcore_barrier()
    frows = NACC // NSUB
    pltpu.sync_copy(acc.at[pl.ds(sid * frows, frows)],
                    out_hbm.at[cid, pl.ds(sid * frows, frows)])


def _make_agg(nck, b0r, b1r, mult):
    return pl.kernel(
        functools.partial(_agg_body, nck, b0r, b1r, mult),
        out_type=jax.ShapeDtypeStruct((NCORE, NACC, DI), jnp.float32),
        mesh=_MESH,
        scratch_types=[
            pltpu.VMEM_SHARED((NACC, DI), jnp.float32),
            pltpu.VMEM((NI, CH), jnp.int32),
            pltpu.VMEM((NI, CH), jnp.int32),
            pltpu.VMEM((CH, DI), jnp.float32),
            pltpu.VMEM((CH, DI), jnp.float32),
            pltpu.SemaphoreType.DMA,
            pltpu.SemaphoreType.DMA,
            pltpu.SemaphoreType.DMA,
            pltpu.SemaphoreType.DMA,
            pltpu.SemaphoreType.DMA,
        ],
    )


_agg_split = _make_agg(EPAD // (NCORE * NSUB) // CH, 0, (EPAD // 2) // CH, 0)
_agg_full = _make_agg(EPAD // NSUB // CH, 0, 0, 1)


def _dot(a, b):
    return jax.lax.dot_general(a, b, (((1,), (0,)), ((), ())),
                               precision=jax.lax.Precision.HIGHEST,
                               preferred_element_type=jnp.float32)


def _dinv_of(deg_blk):
    d = deg_blk[0] + deg_blk[1]
    return lax.rsqrt(d + 1.0)


def _tc1_body(deg_ref, x_ref, hs_ref):
    hs_ref[...] = x_ref[...] * _dinv_of(deg_ref[...])


def _tc2_body(p_ref, hs_ref, deg_ref, w_ref, b_ref, hs2_ref):
    dinv = _dinv_of(deg_ref[...])
    g = (p_ref[0] + p_ref[1] + hs_ref[...]) * dinv
    h = jnp.maximum(_dot(g, w_ref[...]) + b_ref[...], 0.0)
    hs2_ref[0] = h[:, :DI] * dinv
    hs2_ref[1] = h[:, DI:] * dinv


def _tc3_body(q_ref, hs2_ref, deg_ref, w_ref, b_ref, wf_ref, bf_ref,
              out_ref):
    dinv = _dinv_of(deg_ref[...])
    g = jnp.concatenate(
        [(q_ref[0] + hs2_ref[0]) * dinv, (q_ref[1] + hs2_ref[1]) * dinv],
        axis=1)
    h = jnp.maximum(_dot(g, w_ref[...]) + b_ref[...], 0.0)
    out_ref[...] = (jnp.sum(h * wf_ref[...], axis=1, keepdims=True)
                    + bf_ref[...])


_GRID = (N // R,)


def _bs(shape, rowdim):
    def imap(i):
        return tuple(i if d == rowdim else 0 for d in range(len(shape)))
    blk = tuple(R if d == rowdim else s for d, s in enumerate(shape))
    return pl.BlockSpec(blk, imap)


_tc1_call = pl.pallas_call(
    _tc1_body,
    grid=_GRID,
    in_specs=[_bs((2, NACC, DI), 1), _bs((N, DI), 0)],
    out_specs=_bs((N, DI), 0),
    out_shape=jax.ShapeDtypeStruct((N, DI), jnp.float32),
)

_tc2_call = pl.pallas_call(
    _tc2_body,
    grid=_GRID,
    in_specs=[_bs((2, NACC, DI), 1), _bs((N, DI), 0), _bs((2, NACC, DI), 1),
              _bs((DI, DH), -1), _bs((1, DH), -1)],
    out_specs=_bs((2, N, DI), 1),
    out_shape=jax.ShapeDtypeStruct((2, N, DI), jnp.float32),
)

_tc3_call = pl.pallas_call(
    _tc3_body,
    grid=_GRID,
    in_specs=[_bs((2, NACC, DI), 1), _bs((2, N, DI), 1),
              _bs((2, NACC, DI), 1), _bs((DH, DH), -1), _bs((1, DH), -1),
              _bs((1, DH), -1), _bs((1, 1), -1)],
    out_specs=_bs((N, 1), 0),
    out_shape=jax.ShapeDtypeStruct((N, 1), jnp.float32),
)


def kernel(x, edge_index, W1, b1, W2, b2, Wf, bf):
    ei = edge_index.astype(jnp.int32)
    pad = EPAD - E
    src = jnp.concatenate([ei[0], jnp.zeros((pad,), jnp.int32)])
    dst = jnp.concatenate([ei[1], jnp.full((pad,), N, jnp.int32)])
    src = src.reshape(EPAD // CH, CH)
    dst = dst.reshape(EPAD // CH, CH)

    deg = _deg_call(dst)
    hs1 = _tc1_call(deg, x)
    p = _agg_split(hs1, src, dst)
    hs2 = _tc2_call(p, hs1, deg, W1, b1.reshape(1, DH))
    q = _agg_full(hs2.reshape(2 * N, DI), src, dst)
    out = _tc3_call(q, hs2, deg, W2, b2.reshape(1, DH),
                    Wf.reshape(1, DH), bf.reshape(1, 1))
    return out


# speedup vs baseline: 1.0162x; 1.0162x over previous
import functools

import jax
import jax.numpy as jnp
from jax import lax
from jax.experimental import pallas as pl
from jax.experimental.pallas import tpu as pltpu
from jax.experimental.pallas import tpu_sc as plsc

N = 10000
E = 320000
DI = 128
DH = 256
CH = 128
NI = 40
EPAD = 327680
NACC = 10240
NSUB = 16
NCORE = 2
R = 1000

_MESH = plsc.VectorSubcoreMesh(core_axis_name="c", subcore_axis_name="s")
_ZV = functools.partial(jnp.zeros, (16,), jnp.float32)


def _deg_body(dst_hbm, out_hbm, acc, di, ones, sz):
    cid = lax.axis_index("c")
    sid = lax.axis_index("s")
    nck = EPAD // (NCORE * NSUB) // CH
    zrows = NACC // NSUB

    def fill(i, _):
        for c8 in range(DI // 16):
            ones[i, pl.ds(c8 * 16, 16)] = _ZV()
        return 0
    lax.fori_loop(0, CH, fill, 0)

    zd = [pltpu.async_copy(ones, acc.at[pl.ds(sid * zrows + i * CH, CH)], sz)
          for i in range(zrows // CH)]
    row0 = (cid * NSUB + sid) * nck
    pltpu.sync_copy(dst_hbm.at[pl.ds(row0, nck)], di)
    for d in zd:
        d.wait()

    one = jnp.ones((16,), jnp.float32)

    def refill(i, _):
        for c8 in range(DI // 16):
            ones[i, pl.ds(c8 * 16, 16)] = one
        return 0
    lax.fori_loop(0, CH, refill, 0)
    plsc.subcore_barrier()

    FL = 8
    def step(c, _):
        ws = [pltpu.async_copy(ones, acc.at[di.at[c * FL + k]], sz, add=True)
              for k in range(FL)]
        for w in ws:
            w.wait()
        return 0
    lax.fori_loop(0, nck // FL, step, 0)

    plsc.subcore_barrier()
    pltpu.sync_copy(acc.at[pl.ds(sid * zrows, zrows)],
                    out_hbm.at[cid, pl.ds(sid * zrows, zrows)])


_deg_call = pl.kernel(
    _deg_body,
    out_type=jax.ShapeDtypeStruct((NCORE, NACC, DI), jnp.float32),
    mesh=_MESH,
    scratch_types=[
        pltpu.VMEM_SHARED((NACC, DI), jnp.float32),
        pltpu.VMEM((EPAD // (NCORE * NSUB) // CH, CH), jnp.int32),
        pltpu.VMEM((CH, DI), jnp.float32),
        pltpu.SemaphoreType.DMA,
    ],
)


def _agg_body(nck, b0r, b1r, mult, tbl, src_hbm, dst_hbm, out_hbm,
              acc, si, di, ra, rb, sz, sga, sgb, ssa, ssb):
    cid = lax.axis_index("c")
    sid = lax.axis_index("s")
    zrows = NACC // NSUB

    def zrow(i, _):
        for c8 in range(DI // 16):
            ra[i, pl.ds(c8 * 16, 16)] = _ZV()
        return 0
    lax.fori_loop(0, CH, zrow, 0)

    zd = [pltpu.async_copy(ra, acc.at[pl.ds(sid * zrows + i * CH, CH)], sz)
          for i in range(zrows // CH)]
    row0 = jnp.where(cid == 0, b0r, b1r) + sid * nck
    off = cid * (mult * N)
    for d in zd:
        d.wait()
    plsc.subcore_barrier()

    def stage(s, _):
        pltpu.sync_copy(src_hbm.at[pl.ds(row0 + s * NI, NI)], si)
        pltpu.sync_copy(dst_hbm.at[pl.ds(row0 + s * NI, NI)], di)
        if mult:
            def shift(r, _):
                for c8 in range(CH // 16):
                    si[r, pl.ds(c8 * 16, 16)] = (
                        si[r, pl.ds(c8 * 16, 16)] + off)
                return 0
            lax.fori_loop(0, NI, shift, 0)
        da = pltpu.async_copy(tbl.at[si.at[0]], ra, sga)
        wb = None
        for p in range(NI // 2):
            if wb is not None:
                wb.wait()
            db = pltpu.async_copy(tbl.at[si.at[2 * p + 1]], rb, sgb)
            da.wait()
            wa = pltpu.async_copy(ra, acc.at[di.at[2 * p]], ssa, add=True)
            db.wait()
            wb = pltpu.async_copy(rb, acc.at[di.at[2 * p + 1]], ssb, add=True)
            wa.wait()
            if p + 1 < NI // 2:
                da = pltpu.async_copy(tbl.at[si.at[2 * p + 2]], ra, sga)
        wb.wait()
        return 0
    lax.fori_loop(0, nck // NI, stage, 0)

    plsc.subcore_barrier()
    frows = NACC // NSUB
    pltpu.sync_copy(acc.at[pl.ds(sid * frows, frows)],
                    out_hbm.at[cid, pl.ds(sid * frows, frows)])


def _make_agg(nck, b0r, b1r, mult):
    return pl.kernel(
        functools.partial(_agg_body, nck, b0r, b1r, mult),
        out_type=jax.ShapeDtypeStruct((NCORE, NACC, DI), jnp.float32),
        mesh=_MESH,
        scratch_types=[
            pltpu.VMEM_SHARED((NACC, DI), jnp.float32),
            pltpu.VMEM((NI, CH), jnp.int32),
            pltpu.VMEM((NI, CH), jnp.int32),
            pltpu.VMEM((CH, DI), jnp.float32),
            pltpu.VMEM((CH, DI), jnp.float32),
            pltpu.SemaphoreType.DMA,
            pltpu.SemaphoreType.DMA,
            pltpu.SemaphoreType.DMA,
            pltpu.SemaphoreType.DMA,
            pltpu.SemaphoreType.DMA,
        ],
    )


_agg_split = _make_agg(EPAD // (NCORE * NSUB) // CH, 0, (EPAD // 2) // CH, 1)
_agg_full = _make_agg(EPAD // NSUB // CH, 0, 0, 1)


def _dot(a, b):
    return jax.lax.dot_general(a, b, (((1,), (0,)), ((), ())),
                               precision=jax.lax.Precision.HIGHEST,
                               preferred_element_type=jnp.float32)


def _dinv_of(deg_blk):
    d = deg_blk[0] + deg_blk[1]
    return lax.rsqrt(d + 1.0)


def _tc1_body(deg_ref, x_ref, hs_ref):
    v = x_ref[...] * _dinv_of(deg_ref[...])
    hs_ref[0] = v
    hs_ref[1] = v


def _tc2_body(p_ref, hs_ref, deg_ref, w_ref, b_ref, hs2_ref):
    dinv = _dinv_of(deg_ref[...])
    g = (p_ref[0] + p_ref[1] + hs_ref[0]) * dinv
    h = jnp.maximum(_dot(g, w_ref[...]) + b_ref[...], 0.0)
    hs2_ref[0] = h[:, :DI] * dinv
    hs2_ref[1] = h[:, DI:] * dinv


def _tc3_body(q_ref, hs2_ref, deg_ref, w_ref, b_ref, wf_ref, bf_ref,
              out_ref):
    dinv = _dinv_of(deg_ref[...])
    g = jnp.concatenate(
        [(q_ref[0] + hs2_ref[0]) * dinv, (q_ref[1] + hs2_ref[1]) * dinv],
        axis=1)
    h = jnp.maximum(_dot(g, w_ref[...]) + b_ref[...], 0.0)
    out_ref[...] = (jnp.sum(h * wf_ref[...], axis=1, keepdims=True)
                    + bf_ref[...])


_GRID = (N // R,)


def _bs(shape, rowdim):
    def imap(i):
        return tuple(i if d == rowdim else 0 for d in range(len(shape)))
    blk = tuple(R if d == rowdim else s for d, s in enumerate(shape))
    return pl.BlockSpec(blk, imap)


_tc1_call = pl.pallas_call(
    _tc1_body,
    grid=_GRID,
    in_specs=[_bs((2, NACC, DI), 1), _bs((N, DI), 0)],
    out_specs=_bs((2, N, DI), 1),
    out_shape=jax.ShapeDtypeStruct((2, N, DI), jnp.float32),
)

_tc2_call = pl.pallas_call(
    _tc2_body,
    grid=_GRID,
    in_specs=[_bs((2, NACC, DI), 1), _bs((2, N, DI), 1), _bs((2, NACC, DI), 1),
              _bs((DI, DH), -1), _bs((1, DH), -1)],
    out_specs=_bs((2, N, DI), 1),
    out_shape=jax.ShapeDtypeStruct((2, N, DI), jnp.float32),
)

_tc3_call = pl.pallas_call(
    _tc3_body,
    grid=_GRID,
    in_specs=[_bs((2, NACC, DI), 1), _bs((2, N, DI), 1),
              _bs((2, NACC, DI), 1), _bs((DH, DH), -1), _bs((1, DH), -1),
              _bs((1, DH), -1), _bs((1, 1), -1)],
    out_specs=_bs((N, 1), 0),
    out_shape=jax.ShapeDtypeStruct((N, 1), jnp.float32),
)


def kernel(x, edge_index, W1, b1, W2, b2, Wf, bf):
    ei = edge_index.astype(jnp.int32)
    pad = EPAD - E
    src = jnp.concatenate([ei[0], jnp.zeros((pad,), jnp.int32)])
    dst = jnp.concatenate([ei[1], jnp.full((pad,), N, jnp.int32)])
    src = src.reshape(EPAD // CH, CH)
    dst = dst.reshape(EPAD // CH, CH)

    deg = _deg_call(dst)
    hs1 = _tc1_call(deg, x)
    p = _agg_split(hs1.reshape(2 * N, DI), src, dst)
    hs2 = _tc2_call(p, hs1, deg, W1, b1.reshape(1, DH))
    q = _agg_full(hs2.reshape(2 * N, DI), src, dst)
    out = _tc3_call(q, hs2, deg, W2, b2.reshape(1, DH),
                    Wf.reshape(1, DH), bf.reshape(1, 1))
    return out
